# baseline (speedup 1.0000x reference)
import functools

import jax
import jax.numpy as jnp
from jax import lax
from jax.experimental import pallas as pl
from jax.experimental.pallas import tpu as pltpu
from jax.experimental.pallas import tpu_sc as plsc

N = 10000
D = 128
H = D // 2
NC = 2
NS = 16
NW = NC * NS
CK = 128
NCHUNK = 160
NBUF = 4
EPT = NCHUNK * CK
EP = NS * EPT
ROWS_PAD = 10240
RPT = ROWS_PAD // NS
TRASH = N
DEG_PAD = 10240
DEG_EPT = EP // NW

_mesh = plsc.VectorSubcoreMesh(core_axis_name="c", subcore_axis_name="s")


@functools.partial(
    pl.kernel,
    out_type=jax.ShapeDtypeStruct((NW, DEG_PAD), jnp.float32),
    mesh=_mesh,
    scratch_types=[
        pltpu.VMEM((DEG_EPT,), jnp.int32),
        pltpu.VMEM((DEG_PAD,), jnp.float32),
    ],
    compiler_params=pltpu.CompilerParams(needs_layout_passes=False),
)
def _deg_kernel(dst_hbm, out_hbm, idx_v, deg_v):
    c = lax.axis_index("c")
    s = lax.axis_index("s")
    wid = c * NS + s
    pltpu.sync_copy(dst_hbm.at[wid], idx_v)
    zeros = jnp.zeros((16,), jnp.float32)

    def zbody(i, carry):
        deg_v[pl.ds(i * 16, 16)] = zeros
        return carry

    lax.fori_loop(0, DEG_PAD // 16, zbody, 0)
    ones = jnp.ones((16,), jnp.float32)

    def body(i, carry):
        idx = idx_v[pl.ds(i * 16, 16)]
        plsc.addupdate_scatter(deg_v, [idx], ones)
        return carry

    lax.fori_loop(0, DEG_EPT // 16, body, 0)
    pltpu.sync_copy(deg_v, out_hbm.at[wid])


@functools.partial(
    pl.kernel,
    out_type=jax.ShapeDtypeStruct((NC, N, H), jnp.float32),
    mesh=_mesh,
    scratch_types=[
        pltpu.VMEM((NCHUNK, CK), jnp.int32),
        pltpu.VMEM((NCHUNK, CK), jnp.int32),
        pltpu.VMEM((CK, H), jnp.float32),
        pltpu.VMEM((CK, H), jnp.float32),
        pltpu.VMEM((CK, H), jnp.float32),
        pltpu.VMEM((CK, H), jnp.float32),
        pltpu.VMEM_SHARED((ROWS_PAD, H), jnp.float32),
        pltpu.SemaphoreType.DMA,
        pltpu.SemaphoreType.DMA,
        pltpu.SemaphoreType.DMA,
        pltpu.SemaphoreType.DMA,
    ],
    compiler_params=pltpu.CompilerParams(needs_layout_passes=False,
                                         use_tc_tiling_on_sc=False),
)
def _agg_kernel(h_hbm, src_hbm, dst_hbm, zero_hbm, out_hbm,
                src_v, dst_v, st0, st1, st2, st3, acc_sh,
                sem0, sem1, sem2, sem3):
    stages = [st0, st1, st2, st3]
    gsems = [sem0, sem1, sem2, sem3]
    c = lax.axis_index("c")
    s = lax.axis_index("s")
    wid = c * NS + s
    base = s * RPT
    pltpu.sync_copy(src_hbm.at[wid], src_v)
    pltpu.sync_copy(dst_hbm.at[wid], dst_v)
    pltpu.sync_copy(zero_hbm, stages[0])
    for k in range(RPT // CK):
        pltpu.sync_copy(stages[0], acc_sh.at[pl.ds(base + k * CK, CK)])
    plsc.subcore_barrier()

    def body(gi, carry):
        g = gi * NBUF
        descs = [
            pltpu.async_copy(h_hbm.at[src_v.at[g + b]], stages[b], gsems[b])
            for b in range(NBUF)
        ]
        for b in range(NBUF):
            descs[b].wait()
            pltpu.sync_copy(stages[b], acc_sh.at[dst_v.at[g + b]], add=True)
        return carry

    lax.fori_loop(0, NCHUNK // NBUF, body, 0)
    plsc.subcore_barrier()
    last = N - (NS - 1) * RPT

    @pl.when(s < NS - 1)
    def _copy_full():
        pltpu.sync_copy(acc_sh.at[pl.ds(base, RPT)],
                        out_hbm.at[c, pl.ds(base, RPT)])

    @pl.when(s == NS - 1)
    def _copy_last():
        pltpu.sync_copy(acc_sh.at[pl.ds(base, last)],
                        out_hbm.at[c, pl.ds(base, last)])


BLK = 400
GRID = 25


def _split(h):
    return jnp.stack([h[:, :H], h[:, H:]], axis=0)


def _tdis_body(degp_ref, dis_ref):
    deg = jnp.sum(degp_ref[...], axis=0) + 1.0
    dis_ref[...] = lax.rsqrt(deg)[:, None]


_tdis = pl.pallas_call(
    _tdis_body,
    in_specs=[pl.BlockSpec((NW, DEG_PAD), lambda: (0, 0))],
    out_specs=pl.BlockSpec((DEG_PAD, 1), lambda: (0, 0)),
    out_shape=jax.ShapeDtypeStruct((DEG_PAD, 1), jnp.float32),
)


def _t1_body(x_ref, w_ref, dis_ref, h_ref):
    h = jnp.dot(x_ref[...], w_ref[...],
                preferred_element_type=jnp.float32,
                precision=lax.Precision.HIGHEST)
    h_ref[...] = _split(h * dis_ref[...])


_t1 = pl.pallas_call(
    _t1_body,
    grid=(GRID,),
    in_specs=[
        pl.BlockSpec((BLK, D), lambda i: (i, 0)),
        pl.BlockSpec((D, D), lambda i: (0, 0)),
        pl.BlockSpec((BLK, 1), lambda i: (i, 0)),
    ],
    out_specs=pl.BlockSpec((2, BLK, H), lambda i: (0, i, 0)),
    out_shape=jax.ShapeDtypeStruct((2, N, H), jnp.float32),
)


def _tmid_body(p_ref, hp_ref, dis_ref, b_ref, w_ref, out_ref):
    dis = dis_ref[...]
    h_lo = dis * (p_ref[0] + hp_ref[0]) + b_ref[:, :H]
    h_hi = dis * (p_ref[1] + hp_ref[1]) + b_ref[:, H:]
    h_lo = jnp.where(h_lo >= 0, h_lo, 0.01 * h_lo)
    h_hi = jnp.where(h_hi >= 0, h_hi, 0.01 * h_hi)
    hw = (jnp.dot(h_lo, w_ref[:H, :],
                  preferred_element_type=jnp.float32,
                  precision=lax.Precision.HIGHEST)
          + jnp.dot(h_hi, w_ref[H:, :],
                    preferred_element_type=jnp.float32,
                    precision=lax.Precision.HIGHEST))
    out_ref[...] = _split(hw * dis)


_tmid = pl.pallas_call(
    _tmid_body,
    grid=(GRID,),
    in_specs=[
        pl.BlockSpec((NC, BLK, H), lambda i: (0, i, 0)),
        pl.BlockSpec((2, BLK, H), lambda i: (0, i, 0)),
        pl.BlockSpec((BLK, 1), lambda i: (i, 0)),
        pl.BlockSpec((1, D), lambda i: (0, 0)),
        pl.BlockSpec((D, D), lambda i: (0, 0)),
    ],
    out_specs=pl.BlockSpec((2, BLK, H), lambda i: (0, i, 0)),
    out_shape=jax.ShapeDtypeStruct((2, N, H), jnp.float32),
)


def _t4_body(p_ref, zp_ref, dis_ref, bmu_ref, bls_ref, mu_ref, ls_ref):
    dis = dis_ref[...]
    mu_ref[...] = dis * (p_ref[0] + zp_ref[0]) + bmu_ref[...]
    ls_ref[...] = dis * (p_ref[1] + zp_ref[1]) + bls_ref[...]


_t4 = pl.pallas_call(
    _t4_body,
    grid=(GRID,),
    in_specs=[
        pl.BlockSpec((NC, BLK, H), lambda i: (0, i, 0)),
        pl.BlockSpec((2, BLK, H), lambda i: (0, i, 0)),
        pl.BlockSpec((BLK, 1), lambda i: (i, 0)),
        pl.BlockSpec((1, H), lambda i: (0, 0)),
        pl.BlockSpec((1, H), lambda i: (0, 0)),
    ],
    out_specs=[
        pl.BlockSpec((BLK, H), lambda i: (i, 0)),
        pl.BlockSpec((BLK, H), lambda i: (i, 0)),
    ],
    out_shape=[
        jax.ShapeDtypeStruct((N, H), jnp.float32),
        jax.ShapeDtypeStruct((N, H), jnp.float32),
    ],
)


def kernel(x, W1, b1, W2, b2, Wmu, bmu, Wls, bls, edge_index):
    src = edge_index[0].astype(jnp.int32)
    dst = edge_index[1].astype(jnp.int32)
    e = src.shape[0]
    pad = EP - e
    src_p = jnp.concatenate([src, jnp.zeros((pad,), jnp.int32)])
    dst_p = jnp.concatenate([dst, jnp.full((pad,), TRASH, jnp.int32)])
    src4 = jnp.stack([src_p, src_p + N]).reshape(NW, NCHUNK, CK)
    dst4 = jnp.stack([dst_p, dst_p]).reshape(NW, NCHUNK, CK)
    dst2 = dst_p.reshape(NW, DEG_EPT)
    zero_hbm = jnp.zeros((CK, H), jnp.float32)

    degp = _deg_kernel(dst2)
    dis = _tdis(degp)[:N]
    h1s = _t1(x, W1, dis)
    p1 = _agg_kernel(h1s.reshape(2 * N, H), src4, dst4, zero_hbm)
    h2s = _tmid(p1, h1s, dis, b1.reshape(1, D), W2)
    p2 = _agg_kernel(h2s.reshape(2 * N, H), src4, dst4, zero_hbm)
    wcat = jnp.concatenate([Wmu, Wls], axis=1)
    zs = _tmid(p2, h2s, dis, b2.reshape(1, D), wcat)
    p3 = _agg_kernel(zs.reshape(2 * N, H), src4, dst4, zero_hbm)
    mu, logstd = _t4(p3, zs, dis, bmu.reshape(1, H), bls.reshape(1, H))
    return (mu, logstd)


# speedup vs baseline: 1.0978x; 1.0978x over previous
import functools

import jax
import jax.numpy as jnp
from jax import lax
from jax.experimental import pallas as pl
from jax.experimental.pallas import tpu as pltpu
from jax.experimental.pallas import tpu_sc as plsc

N = 10000
D = 128
H = D // 2
NC = 2
NS = 16
NW = NC * NS
CK = 128
NCHUNK = 160
NBUF = 5
EPT = NCHUNK * CK
EP = NS * EPT
ROWS_PAD = 10240
RPT = ROWS_PAD // NS
TRASH = N
DEG_PAD = 10240
DEG_EPT = EP // NW

_mesh = plsc.VectorSubcoreMesh(core_axis_name="c", subcore_axis_name="s")


@functools.partial(
    pl.kernel,
    out_type=jax.ShapeDtypeStruct((NW, DEG_PAD), jnp.float32),
    mesh=_mesh,
    scratch_types=[
        pltpu.VMEM((DEG_EPT,), jnp.int32),
        pltpu.VMEM((DEG_PAD,), jnp.float32),
    ],
    compiler_params=pltpu.CompilerParams(needs_layout_passes=False),
)
def _deg_kernel(dst_hbm, out_hbm, idx_v, deg_v):
    c = lax.axis_index("c")
    s = lax.axis_index("s")
    wid = c * NS + s
    pltpu.sync_copy(dst_hbm.at[wid], idx_v)
    zeros = jnp.zeros((16,), jnp.float32)

    def zbody(i, carry):
        deg_v[pl.ds(i * 16, 16)] = zeros
        return carry

    lax.fori_loop(0, DEG_PAD // 16, zbody, 0)
    ones = jnp.ones((16,), jnp.float32)

    def body(i, carry):
        idx = idx_v[pl.ds(i * 16, 16)]
        plsc.addupdate_scatter(deg_v, [idx], ones)
        return carry

    lax.fori_loop(0, DEG_EPT // 16, body, 0)
    pltpu.sync_copy(deg_v, out_hbm.at[wid])


@functools.partial(
    pl.kernel,
    out_type=jax.ShapeDtypeStruct((NC, N, H), jnp.float32),
    mesh=_mesh,
    scratch_types=[
        pltpu.VMEM((NCHUNK, CK), jnp.int32),
        pltpu.VMEM((NCHUNK, CK), jnp.int32),
        pltpu.VMEM((CK, H), jnp.float32),
        pltpu.VMEM((CK, H), jnp.float32),
        pltpu.VMEM((CK, H), jnp.float32),
        pltpu.VMEM((CK, H), jnp.float32),
        pltpu.VMEM((CK, H), jnp.float32),
        pltpu.VMEM_SHARED((ROWS_PAD, H), jnp.float32),
        pltpu.SemaphoreType.DMA,
        pltpu.SemaphoreType.DMA,
        pltpu.SemaphoreType.DMA,
        pltpu.SemaphoreType.DMA,
        pltpu.SemaphoreType.DMA,
        pltpu.SemaphoreType.DMA,
        pltpu.SemaphoreType.DMA,
        pltpu.SemaphoreType.DMA,
        pltpu.SemaphoreType.DMA,
        pltpu.SemaphoreType.DMA,
    ],
    compiler_params=pltpu.CompilerParams(needs_layout_passes=False,
                                         use_tc_tiling_on_sc=False),
)
def _agg_kernel(h_hbm, src_hbm, dst_hbm, zero_hbm, out_hbm,
                src_v, dst_v, st0, st1, st2, st3, st4,
                acc_sh, ga0, ga1, ga2, ga3, ga4,
                sa0, sa1, sa2, sa3, sa4):
    stages = [st0, st1, st2, st3, st4]
    gsems = [ga0, ga1, ga2, ga3, ga4]
    ssems = [sa0, sa1, sa2, sa3, sa4]
    c = lax.axis_index("c")
    s = lax.axis_index("s")
    wid = c * NS + s
    base = s * RPT
    pltpu.sync_copy(src_hbm.at[wid], src_v)
    pltpu.sync_copy(dst_hbm.at[wid], dst_v)
    pltpu.sync_copy(zero_hbm, stages[0])
    for k in range(RPT // CK):
        pltpu.sync_copy(stages[0], acc_sh.at[pl.ds(base + k * CK, CK)])
    plsc.subcore_barrier()

    for b in range(NBUF):
        pltpu.async_copy(h_hbm.at[src_v.at[b]], stages[b], gsems[b])

    def body(gi, carry):
        g = gi * NBUF
        for b in range(NBUF):
            pltpu.make_async_copy(
                h_hbm.at[src_v.at[g + b]], stages[b], gsems[b]).wait()
            pltpu.async_copy(
                stages[b], acc_sh.at[dst_v.at[g + b]], ssems[b], add=True)
        for b in range(NBUF):
            pltpu.make_async_copy(
                stages[b], acc_sh.at[dst_v.at[g + b]], ssems[b]).wait()
            nxt = lax.rem(g + b + NBUF, NCHUNK)
            pltpu.async_copy(h_hbm.at[src_v.at[nxt]], stages[b], gsems[b])
        return carry

    lax.fori_loop(0, NCHUNK // NBUF, body, 0)
    for b in range(NBUF):
        pltpu.make_async_copy(
            h_hbm.at[src_v.at[b]], stages[b], gsems[b]).wait()
    plsc.subcore_barrier()
    last = N - (NS - 1) * RPT

    @pl.when(s < NS - 1)
    def _copy_full():
        pltpu.sync_copy(acc_sh.at[pl.ds(base, RPT)],
                        out_hbm.at[c, pl.ds(base, RPT)])

    @pl.when(s == NS - 1)
    def _copy_last():
        pltpu.sync_copy(acc_sh.at[pl.ds(base, last)],
                        out_hbm.at[c, pl.ds(base, last)])


BLK = 400
GRID = 25


def _split(h):
    return jnp.stack([h[:, :H], h[:, H:]], axis=0)


def _tdis_body(degp_ref, dis_ref):
    deg = jnp.sum(degp_ref[...], axis=0) + 1.0
    dis_ref[...] = lax.rsqrt(deg)[:, None]


_tdis = pl.pallas_call(
    _tdis_body,
    in_specs=[pl.BlockSpec((NW, DEG_PAD), lambda: (0, 0))],
    out_specs=pl.BlockSpec((DEG_PAD, 1), lambda: (0, 0)),
    out_shape=jax.ShapeDtypeStruct((DEG_PAD, 1), jnp.float32),
)


def _t1_body(x_ref, w_ref, dis_ref, h_ref):
    h = jnp.dot(x_ref[...], w_ref[...],
                preferred_element_type=jnp.float32,
                precision=lax.Precision.HIGHEST)
    h_ref[...] = _split(h * dis_ref[...])


_t1 = pl.pallas_call(
    _t1_body,
    grid=(GRID,),
    in_specs=[
        pl.BlockSpec((BLK, D), lambda i: (i, 0)),
        pl.BlockSpec((D, D), lambda i: (0, 0)),
        pl.BlockSpec((BLK, 1), lambda i: (i, 0)),
    ],
    out_specs=pl.BlockSpec((2, BLK, H), lambda i: (0, i, 0)),
    out_shape=jax.ShapeDtypeStruct((2, N, H), jnp.float32),
)


def _tmid_body(p_ref, hp_ref, dis_ref, b_ref, w_ref, out_ref):
    dis = dis_ref[...]
    h_lo = dis * (p_ref[0] + hp_ref[0]) + b_ref[:, :H]
    h_hi = dis * (p_ref[1] + hp_ref[1]) + b_ref[:, H:]
    h_lo = jnp.where(h_lo >= 0, h_lo, 0.01 * h_lo)
    h_hi = jnp.where(h_hi >= 0, h_hi, 0.01 * h_hi)
    hw = (jnp.dot(h_lo, w_ref[:H, :],
                  preferred_element_type=jnp.float32,
                  precision=lax.Precision.HIGHEST)
          + jnp.dot(h_hi, w_ref[H:, :],
                    preferred_element_type=jnp.float32,
                    precision=lax.Precision.HIGHEST))
    out_ref[...] = _split(hw * dis)


_tmid = pl.pallas_call(
    _tmid_body,
    grid=(GRID,),
    in_specs=[
        pl.BlockSpec((NC, BLK, H), lambda i: (0, i, 0)),
        pl.BlockSpec((2, BLK, H), lambda i: (0, i, 0)),
        pl.BlockSpec((BLK, 1), lambda i: (i, 0)),
        pl.BlockSpec((1, D), lambda i: (0, 0)),
        pl.BlockSpec((D, D), lambda i: (0, 0)),
    ],
    out_specs=pl.BlockSpec((2, BLK, H), lambda i: (0, i, 0)),
    out_shape=jax.ShapeDtypeStruct((2, N, H), jnp.float32),
)


def _t4_body(p_ref, zp_ref, dis_ref, bmu_ref, bls_ref, mu_ref, ls_ref):
    dis = dis_ref[...]
    mu_ref[...] = dis * (p_ref[0] + zp_ref[0]) + bmu_ref[...]
    ls_ref[...] = dis * (p_ref[1] + zp_ref[1]) + bls_ref[...]


_t4 = pl.pallas_call(
    _t4_body,
    grid=(GRID,),
    in_specs=[
        pl.BlockSpec((NC, BLK, H), lambda i: (0, i, 0)),
        pl.BlockSpec((2, BLK, H), lambda i: (0, i, 0)),
        pl.BlockSpec((BLK, 1), lambda i: (i, 0)),
        pl.BlockSpec((1, H), lambda i: (0, 0)),
        pl.BlockSpec((1, H), lambda i: (0, 0)),
    ],
    out_specs=[
        pl.BlockSpec((BLK, H), lambda i: (i, 0)),
        pl.BlockSpec((BLK, H), lambda i: (i, 0)),
    ],
    out_shape=[
        jax.ShapeDtypeStruct((N, H), jnp.float32),
        jax.ShapeDtypeStruct((N, H), jnp.float32),
    ],
)


def kernel(x, W1, b1, W2, b2, Wmu, bmu, Wls, bls, edge_index):
    src = edge_index[0].astype(jnp.int32)
    dst = edge_index[1].astype(jnp.int32)
    e = src.shape[0]
    pad = EP - e
    src_p = jnp.concatenate([src, jnp.zeros((pad,), jnp.int32)])
    dst_p = jnp.concatenate([dst, jnp.full((pad,), TRASH, jnp.int32)])
    src4 = jnp.stack([src_p, src_p + N]).reshape(NW, NCHUNK, CK)
    dst4 = jnp.stack([dst_p, dst_p]).reshape(NW, NCHUNK, CK)
    dst2 = dst_p.reshape(NW, DEG_EPT)
    zero_hbm = jnp.zeros((CK, H), jnp.float32)

    degp = _deg_kernel(dst2)
    dis = _tdis(degp)[:N]
    h1s = _t1(x, W1, dis)
    p1 = _agg_kernel(h1s.reshape(2 * N, H), src4, dst4, zero_hbm)
    h2s = _tmid(p1, h1s, dis, b1.reshape(1, D), W2)
    p2 = _agg_kernel(h2s.reshape(2 * N, H), src4, dst4, zero_hbm)
    wcat = jnp.concatenate([Wmu, Wls], axis=1)
    zs = _tmid(p2, h2s, dis, b2.reshape(1, D), wcat)
    p3 = _agg_kernel(zs.reshape(2 * N, H), src4, dst4, zero_hbm)
    mu, logstd = _t4(p3, zs, dis, bmu.reshape(1, H), bls.reshape(1, H))
    return (mu, logstd)
